# Initial kernel scaffold; baseline (speedup 1.0000x reference)
import jax
import jax.numpy as jnp
from jax.experimental import pallas as pl


def kernel(feature, same_type_adj, W, b):
    raise NotImplementedError("write your pallas kernel here")


# speedup vs baseline: 1.3784x; 1.3784x over previous
import functools

import jax
import jax.numpy as jnp
from jax.experimental import pallas as pl
from jax.experimental.pallas import tpu as pltpu

N = 4096
D = 256
R = 3
ROW_TILE = 256


def _fused_body(feature_ref, adj_ref, w_ref, out_ref, h_ref):
    i = pl.program_id(0)

    @pl.when(i == 0)
    def _compute_h():
        h_ref[...] = jnp.tanh(
            jnp.dot(feature_ref[...], w_ref[...],
                    preferred_element_type=jnp.float32))

    a = adj_ref[0] + adj_ref[1] + adj_ref[2]
    out_ref[...] = jnp.dot(a, h_ref[...], preferred_element_type=jnp.float32)


@jax.jit
def kernel(feature, same_type_adj, W, b):
    del b
    grid = (N // ROW_TILE,)
    return pl.pallas_call(
        _fused_body,
        grid=grid,
        in_specs=[
            pl.BlockSpec((N, D), lambda i: (0, 0)),
            pl.BlockSpec((R, ROW_TILE, N), lambda i: (0, i, 0)),
            pl.BlockSpec((D, D), lambda i: (0, 0)),
        ],
        out_specs=pl.BlockSpec((ROW_TILE, D), lambda i: (i, 0)),
        out_shape=jax.ShapeDtypeStruct((N, D), jnp.float32),
        scratch_shapes=[pltpu.VMEM((N, D), jnp.float32)],
    )(feature, same_type_adj, W)


# speedup vs baseline: 1.3987x; 1.0147x over previous
import functools

import jax
import jax.numpy as jnp
from jax.experimental import pallas as pl
from jax.experimental.pallas import tpu as pltpu

N = 4096
D = 256
R = 3
ROW_TILE = 128


def _fused_body(feature_ref, adj_ref, w_ref, out_ref, h_ref):
    i = pl.program_id(0)

    @pl.when(i == 0)
    def _compute_h():
        h_ref[...] = jnp.tanh(
            jnp.dot(feature_ref[...], w_ref[...],
                    preferred_element_type=jnp.float32))

    a = adj_ref[0] + adj_ref[1] + adj_ref[2]
    out_ref[...] = jnp.dot(a, h_ref[...], preferred_element_type=jnp.float32)


@jax.jit
def kernel(feature, same_type_adj, W, b):
    del b
    grid = (N // ROW_TILE,)
    return pl.pallas_call(
        _fused_body,
        grid=grid,
        in_specs=[
            pl.BlockSpec((N, D), lambda i: (0, 0)),
            pl.BlockSpec((R, ROW_TILE, N), lambda i: (0, i, 0)),
            pl.BlockSpec((D, D), lambda i: (0, 0)),
        ],
        out_specs=pl.BlockSpec((ROW_TILE, D), lambda i: (i, 0)),
        out_shape=jax.ShapeDtypeStruct((N, D), jnp.float32),
        scratch_shapes=[pltpu.VMEM((N, D), jnp.float32)],
    )(feature, same_type_adj, W)
